# Initial kernel scaffold; baseline (speedup 1.0000x reference)
import functools

import jax
import jax.numpy as jnp
from jax import lax
from jax.experimental import pallas as pl
from jax.experimental.pallas import tpu as pltpu
from jax.experimental.pallas import tpu_sc as plsc

B, N = 1024, 50
E = B * N
NC, NS, L = 2, 16, 16
NW = NC * NS
EPW = E // NW
GROUPS = EPW // L
F = 7
DOUT = 28


def _body(info_hbm, mask_hbm, w_hbm, b_hbm, az_hbm, tz_hbm, out_hbm,
          info_v, mask_v, w_v, b_v, az_v, tz_v, out_v):
  wid = lax.axis_index("s") * NC + lax.axis_index("c")
  ebase = wid * EPW

  pltpu.sync_copy(info_hbm.at[pl.ds(ebase * F, EPW * F)], info_v)
  pltpu.sync_copy(mask_hbm.at[pl.ds(ebase, EPW)], mask_v)
  pltpu.sync_copy(w_hbm, w_v)
  pltpu.sync_copy(b_hbm, b_v)
  pltpu.sync_copy(az_hbm, az_v)
  pltpu.sync_copy(tz_hbm, tz_v)

  lane = lax.iota(jnp.int32, L)
  lane_f = lane * F
  lane_o = lane * DOUT

  def group(g, carry):
    e0 = g * L
    m = mask_v[pl.ds(e0, L)]
    fb = e0 * F
    f = [plsc.load_gather(info_v, [lane_f + (fb + k)]) * m for k in range(F)]

    feat = []
    for j in range(16):
      acc = f[0] * w_v[5 * j]
      for k in range(1, 5):
        acc = acc + f[k] * w_v[5 * j + k]
      feat.append(jnp.maximum(acc + b_v[j], 0.0))

    z = f[5].astype(jnp.int32)
    t = f[6].astype(jnp.int32)
    valid = m >= 0.5
    cond = valid & (z >= 1) & (z <= 94)
    fm = jnp.where(valid, m, 0.0)
    cm = jnp.where(cond, m, 0.0)
    zi = jnp.clip(z, 0, 94) * 8
    ti = jnp.clip(t, 0, 5) * 4

    emb = [plsc.load_gather(az_v, [zi + d]) * cm for d in range(8)]
    emb += [plsc.load_gather(tz_v, [ti + d]) * cm for d in range(4)]

    ob = e0 * DOUT
    for ch in range(16):
      plsc.store_scatter(out_v, [lane_o + (ob + ch)], feat[ch] * fm)
    for d in range(12):
      plsc.store_scatter(out_v, [lane_o + (ob + 16 + d)], emb[d])
    return carry

  lax.fori_loop(0, GROUPS, group, 0)
  pltpu.sync_copy(out_v, out_hbm.at[pl.ds(ebase * DOUT, EPW * DOUT)])


_sc_call = functools.partial(
    pl.kernel,
    out_type=jax.ShapeDtypeStruct((E * DOUT,), jnp.float32),
    mesh=plsc.VectorSubcoreMesh(core_axis_name="c", subcore_axis_name="s"),
    scratch_types=[
        pltpu.VMEM((EPW * F,), jnp.float32),
        pltpu.VMEM((EPW,), jnp.float32),
        pltpu.VMEM((80,), jnp.float32),
        pltpu.VMEM((16,), jnp.float32),
        pltpu.VMEM((768,), jnp.float32),
        pltpu.VMEM((32,), jnp.float32),
    ],
)(_body)


@jax.jit
def kernel(elements_info, elements_mask, W, b, atom_embedding, type_embedding):
  info_f = elements_info.reshape(-1)
  mask_f = elements_mask.reshape(-1)
  w_f = W.reshape(-1)
  az_f = jnp.pad(atom_embedding.reshape(-1), (0, 8))
  tz_f = jnp.pad(type_embedding.reshape(-1), (0, 8))
  out = _sc_call(info_f, mask_f, w_f, b, az_f, tz_f)
  return out.reshape(B, N, DOUT)


# speedup vs baseline: 3.8557x; 3.8557x over previous
import functools

import jax
import jax.numpy as jnp
from jax import lax
from jax.experimental import pallas as pl
from jax.experimental.pallas import tpu as pltpu
from jax.experimental.pallas import tpu_sc as plsc

B, N = 1024, 50
E = B * N
NC, NS, L = 2, 16, 16
NW = NC * NS
EPW = E // NW
GROUPS = EPW // L
F = 7
DOUT = 28


def _body(info_hbm, mask_hbm, w_hbm, b_hbm, az_hbm, tz_hbm, out_hbm,
          info_v, mask_v, w_v, b_v, az_v, tz_v, out_v):
  wid = lax.axis_index("s") * NC + lax.axis_index("c")
  ebase = wid * EPW

  pltpu.sync_copy(info_hbm.at[pl.ds(ebase * F, EPW * F)], info_v)
  pltpu.sync_copy(mask_hbm.at[pl.ds(ebase, EPW)], mask_v)
  pltpu.sync_copy(w_hbm, w_v)
  pltpu.sync_copy(b_hbm, b_v)
  pltpu.sync_copy(az_hbm, az_v)
  pltpu.sync_copy(tz_hbm, tz_v)

  lane = lax.iota(jnp.int32, L)
  lane_f = lane * F
  lane_o = lane * DOUT

  wchunk = [w_v[pl.ds(16 * i, 16)] for i in range(5)]
  bchunk = b_v[pl.ds(0, 16)]
  ws = [wchunk[i // 16][i % 16] for i in range(80)]
  bs = [bchunk[j] for j in range(16)]

  def group(g, carry):
    e0 = g * L
    m = mask_v[pl.ds(e0, L)]
    fb = e0 * F
    f = [plsc.load_gather(info_v, [lane_f + (fb + k)]) * m for k in range(F)]

    feat = []
    for j in range(16):
      acc = f[0] * ws[5 * j]
      for k in range(1, 5):
        acc = acc + f[k] * ws[5 * j + k]
      feat.append(jnp.maximum(acc + bs[j], 0.0))

    z = f[5].astype(jnp.int32)
    t = f[6].astype(jnp.int32)
    valid = m >= 0.5
    cond = valid & (z >= 1) & (z <= 94)
    fm = jnp.where(valid, m, 0.0)
    cm = jnp.where(cond, m, 0.0)
    zi = jnp.clip(z, 0, 94) * 8
    ti = jnp.clip(t, 0, 5) * 4

    emb = [plsc.load_gather(az_v, [zi + d]) * cm for d in range(8)]
    emb += [plsc.load_gather(tz_v, [ti + d]) * cm for d in range(4)]

    ob = e0 * DOUT
    for ch in range(16):
      plsc.store_scatter(out_v, [lane_o + (ob + ch)], feat[ch] * fm)
    for d in range(12):
      plsc.store_scatter(out_v, [lane_o + (ob + 16 + d)], emb[d])
    return carry

  lax.fori_loop(0, GROUPS, group, 0)
  pltpu.sync_copy(out_v, out_hbm.at[pl.ds(ebase * DOUT, EPW * DOUT)])


_sc_call = functools.partial(
    pl.kernel,
    out_type=jax.ShapeDtypeStruct((E * DOUT,), jnp.float32),
    mesh=plsc.VectorSubcoreMesh(core_axis_name="c", subcore_axis_name="s"),
    compiler_params=pltpu.CompilerParams(needs_layout_passes=False),
    scratch_types=[
        pltpu.VMEM((EPW * F,), jnp.float32),
        pltpu.VMEM((EPW,), jnp.float32),
        pltpu.VMEM((80,), jnp.float32),
        pltpu.VMEM((16,), jnp.float32),
        pltpu.VMEM((768,), jnp.float32),
        pltpu.VMEM((32,), jnp.float32),
        pltpu.VMEM((EPW * DOUT,), jnp.float32),
    ],
)(_body)


@jax.jit
def kernel(elements_info, elements_mask, W, b, atom_embedding, type_embedding):
  info_f = elements_info.reshape(-1)
  mask_f = elements_mask.reshape(-1)
  w_f = W.reshape(-1)
  az_f = jnp.pad(atom_embedding.reshape(-1), (0, 8))
  tz_f = jnp.pad(type_embedding.reshape(-1), (0, 8))
  out = _sc_call(info_f, mask_f, w_f, b, az_f, tz_f)
  return out.reshape(B, N, DOUT)


# speedup vs baseline: 3.8584x; 1.0007x over previous
import functools

import jax
import jax.numpy as jnp
from jax import lax
from jax.experimental import pallas as pl
from jax.experimental.pallas import tpu as pltpu
from jax.experimental.pallas import tpu_sc as plsc

B, N = 1024, 50
E = B * N
NC, NS, L = 2, 16, 16
NW = NC * NS
EPW = E // NW
GROUPS = EPW // L
F = 7
DOUT = 28


def _body(info_hbm, mask_hbm, w_hbm, b_hbm, az_hbm, tz_hbm, out_hbm,
          info_v, mask_v, w_v, b_v, az_v, tz_v, out_v):
  wid = lax.axis_index("s") * NC + lax.axis_index("c")
  ebase = wid * EPW

  pltpu.sync_copy(info_hbm.at[pl.ds(ebase * F, EPW * F)], info_v)
  pltpu.sync_copy(mask_hbm.at[pl.ds(ebase, EPW)], mask_v)
  pltpu.sync_copy(w_hbm, w_v)
  pltpu.sync_copy(b_hbm, b_v)
  pltpu.sync_copy(az_hbm, az_v)
  pltpu.sync_copy(tz_hbm, tz_v)

  lane = lax.iota(jnp.int32, L)
  lane_f = lane * F
  lane_o = lane * DOUT

  wchunk = [w_v[pl.ds(16 * i, 16)] for i in range(5)]
  bchunk = b_v[pl.ds(0, 16)]
  ws = [wchunk[i // 16][i % 16] for i in range(80)]
  bs = [bchunk[j] for j in range(16)]

  def group(g, carry):
    e0 = g * L
    m = mask_v[pl.ds(e0, L)]
    fb = e0 * F
    f = [plsc.load_gather(info_v, [lane_f + (fb + k)]) * m for k in range(F)]

    feat = []
    for j in range(16):
      acc = f[0] * ws[5 * j]
      for k in range(1, 5):
        acc = acc + f[k] * ws[5 * j + k]
      feat.append(jnp.maximum(acc + bs[j], 0.0))

    z = f[5].astype(jnp.int32)
    t = f[6].astype(jnp.int32)
    valid = m >= 0.5
    cond = valid & (z >= 1) & (z <= 94)
    fm = jnp.where(valid, m, 0.0)
    cm = jnp.where(cond, m, 0.0)
    zi = jnp.clip(z, 0, 94) * 8
    ti = jnp.clip(t, 0, 5) * 4

    emb = [plsc.load_gather(az_v, [zi + d]) * cm for d in range(8)]
    emb += [plsc.load_gather(tz_v, [ti + d]) * cm for d in range(4)]

    ob = e0 * DOUT
    for ch in range(16):
      plsc.store_scatter(out_v, [lane_o + (ob + ch)], feat[ch] * fm)
    for d in range(12):
      plsc.store_scatter(out_v, [lane_o + (ob + 16 + d)], emb[d])
    return carry

  lax.fori_loop(0, GROUPS, group, 0)
  pltpu.sync_copy(out_v, out_hbm.at[pl.ds(ebase * DOUT, EPW * DOUT)])


_sc_call = functools.partial(
    pl.kernel,
    out_type=jax.ShapeDtypeStruct((E * DOUT,), jnp.float32),
    mesh=plsc.VectorSubcoreMesh(core_axis_name="c", subcore_axis_name="s"),
    compiler_params=pltpu.CompilerParams(needs_layout_passes=False),
    scratch_types=[
        pltpu.VMEM((EPW * F,), jnp.float32),
        pltpu.VMEM((EPW,), jnp.float32),
        pltpu.VMEM((80,), jnp.float32),
        pltpu.VMEM((16,), jnp.float32),
        pltpu.VMEM((760,), jnp.float32),
        pltpu.VMEM((24,), jnp.float32),
        pltpu.VMEM((EPW * DOUT,), jnp.float32),
    ],
)(_body)


@jax.jit
def kernel(elements_info, elements_mask, W, b, atom_embedding, type_embedding):
  info_f = elements_info.reshape(-1)
  mask_f = elements_mask.reshape(-1)
  w_f = W.reshape(-1)
  az_f = atom_embedding.reshape(-1)
  tz_f = type_embedding.reshape(-1)
  out = _sc_call(info_f, mask_f, w_f, b, az_f, tz_f)
  return out.reshape(B, N, DOUT)
